# baseline (speedup 1.0000x reference)
import functools

import jax
import jax.numpy as jnp
from jax import lax
from jax.experimental import pallas as pl
from jax.experimental.pallas import tpu as pltpu
from jax.experimental.pallas import tpu_sc as plsc

NC = 2
NS = 16
L = 16

C = 40
NPHASE = 4
NCHUNK = 64
EPT = C * NCHUNK * NPHASE
E_PAD = NC * NS * EPT
ACC_ROWS = 10240


def _sc_aggregate(x, src, dst, n_nodes, d_feat):
  rows_per_tile = ACC_ROWS // NS

  mesh = plsc.VectorSubcoreMesh(
      core_axis_name="c", subcore_axis_name="s", num_cores=NC)

  @functools.partial(
      pl.kernel,
      mesh=mesh,
      out_type=(
          jax.ShapeDtypeStruct((NC, ACC_ROWS, d_feat), jnp.float32),
          jax.ShapeDtypeStruct((NC, ACC_ROWS, d_feat), jnp.float32),
      ),
      scratch_types=[
          pltpu.VMEM((NCHUNK, C), jnp.int32),
          pltpu.VMEM((NCHUNK, C), jnp.int32),
          pltpu.VMEM((C, d_feat), jnp.float32),
          pltpu.VMEM((C, d_feat), jnp.float32),
          pltpu.VMEM((C, d_feat), jnp.float32),
          pltpu.VMEM((C, d_feat), jnp.float32),
          pltpu.VMEM_SHARED((ACC_ROWS, d_feat), jnp.float32),
          [pltpu.SemaphoreType.DMA] * 4,
          [pltpu.SemaphoreType.DMA] * 4,
      ],
  )
  def k(x_hbm, src_hbm, dst_hbm, sums_hbm, cnts_hbm,
        src_v, dst_v, rows0, rows1, rows2, rows3, acc_sh, gsems, ssems):
    cid = lax.axis_index("c")
    sid = lax.axis_index("s")
    abase = sid * rows_per_tile

    def fill_rows0(val):
      @pl.loop(0, C)
      def _(r):
        for cc in range(d_feat // L):
          rows0[r, pl.ds(cc * L, L)] = jnp.full((L,), val, jnp.float32)

    def zero_acc():
      for b in range(rows_per_tile // C):
        pltpu.sync_copy(rows0, acc_sh.at[pl.ds(abase + b * C, C)])

    fill_rows0(0.0)
    zero_acc()

    plsc.subcore_barrier()

    bufs = (rows0, rows1, rows2, rows3)

    def gather_start(cix, b):
      pltpu.async_copy(x_hbm.at[src_v.at[cix]], bufs[b], gsems[b])

    def gather_wait(cix, b):
      pltpu.make_async_copy(x_hbm.at[src_v.at[cix]], bufs[b], gsems[b]).wait()

    def scatter_start(cix, b):
      pltpu.async_copy(bufs[b], acc_sh.at[dst_v.at[cix]], ssems[b], add=True)

    def scatter_wait(cix, b):
      pltpu.make_async_copy(
          bufs[b], acc_sh.at[dst_v.at[cix]], ssems[b]).wait()

    blk = (cid * NS + sid) * NPHASE

    for p in range(NPHASE):
      pltpu.sync_copy(dst_hbm.at[blk + p], dst_v)
      pltpu.sync_copy(src_hbm.at[blk + p], src_v)

      gather_start(0, 0)
      gather_start(1, 1)

      @pl.loop(0, NCHUNK, step=4)
      def _(j):
        for b in range(4):
          cix = j + b
          b2 = (b + 2) % 4
          nxt = cix + 2

          @pl.when(nxt < NCHUNK)
          def _():
            @pl.when(cix >= 2)
            def _():
              scatter_wait(cix - 2, b2)

            gather_start(nxt, b2)

          gather_wait(cix, b)
          scatter_start(cix, b)

      for b in range(4):
        scatter_wait(NCHUNK - 4 + b, b)

    plsc.subcore_barrier()

    pltpu.sync_copy(acc_sh.at[pl.ds(abase, rows_per_tile)],
                    sums_hbm.at[cid, pl.ds(abase, rows_per_tile)])
    fill_rows0(0.0)
    zero_acc()
    plsc.subcore_barrier()

    fill_rows0(1.0)
    for p in range(NPHASE):
      pltpu.sync_copy(dst_hbm.at[blk + p], dst_v)

      @pl.loop(0, NCHUNK)
      def _(j):
        pltpu.sync_copy(rows0, acc_sh.at[dst_v.at[j]], add=True)

    plsc.subcore_barrier()

    pltpu.sync_copy(acc_sh.at[pl.ds(abase, rows_per_tile)],
                    cnts_hbm.at[cid, pl.ds(abase, rows_per_tile)])

  return k(x, src, dst)


def _finalize(agg, n_nodes, d_feat):
  rb = 1000
  grid = n_nodes // rb

  def body(s_ref, c_ref, o_ref):
    cnt = c_ref[0, :, 0:1] + c_ref[1, :, 0:1]
    o_ref[...] = (s_ref[0] + s_ref[1]) / jnp.maximum(cnt, 1.0)

  return pl.pallas_call(
      body,
      grid=(grid,),
      in_specs=[pl.BlockSpec((NC, rb, d_feat), lambda i: (0, i, 0)),
                pl.BlockSpec((NC, rb, d_feat), lambda i: (0, i, 0))],
      out_specs=pl.BlockSpec((rb, d_feat), lambda i: (i, 0)),
      out_shape=jax.ShapeDtypeStruct((n_nodes, d_feat), jnp.float32),
  )(*agg)


@jax.jit
def kernel(x, edge_index):
  n_nodes, d_feat = x.shape
  n_edges = edge_index.shape[1]

  ei = edge_index.astype(jnp.int32)
  pad = E_PAD - n_edges
  src = jnp.concatenate([ei[0], jnp.zeros((pad,), jnp.int32)])
  dst = jnp.concatenate([ei[1], jnp.full((pad,), n_nodes, jnp.int32)])
  src = src.reshape(NC * NS * NPHASE, NCHUNK, C)
  dst = dst.reshape(NC * NS * NPHASE, NCHUNK, C)

  agg = _sc_aggregate(x, src, dst, n_nodes, d_feat)
  return _finalize(agg, n_nodes, d_feat)


# speedup vs baseline: 1.0849x; 1.0849x over previous
import functools

import jax
import jax.numpy as jnp
from jax import lax
from jax.experimental import pallas as pl
from jax.experimental.pallas import tpu as pltpu
from jax.experimental.pallas import tpu_sc as plsc

NC = 2
NS = 16
L = 16

C = 40
NPHASE = 8
NCHUNK = 64
EPT = C * NCHUNK * NPHASE
E_PAD = NS * EPT
ACC_ROWS = 10240


def _sc_aggregate(x, src, dst, n_nodes, d_feat):
  rows_per_tile = ACC_ROWS // NS

  mesh = plsc.VectorSubcoreMesh(
      core_axis_name="c", subcore_axis_name="s", num_cores=NC)

  @functools.partial(
      pl.kernel,
      mesh=mesh,
      out_type=(
          jax.ShapeDtypeStruct((ACC_ROWS, d_feat), jnp.float32),
          jax.ShapeDtypeStruct((ACC_ROWS, d_feat), jnp.float32),
      ),
      scratch_types=[
          pltpu.VMEM((NCHUNK, C), jnp.int32),
          pltpu.VMEM((NCHUNK, C), jnp.int32),
          pltpu.VMEM((C, d_feat), jnp.float32),
          pltpu.VMEM((C, d_feat), jnp.float32),
          pltpu.VMEM((C, d_feat), jnp.float32),
          pltpu.VMEM((C, d_feat), jnp.float32),
          pltpu.VMEM_SHARED((ACC_ROWS, d_feat), jnp.float32),
          [pltpu.SemaphoreType.DMA] * 4,
          [pltpu.SemaphoreType.DMA] * 4,
      ],
  )
  def k(x_hbm, src_hbm, dst_hbm, sums_hbm, cnts_hbm,
        src_v, dst_v, rows0, rows1, rows2, rows3, acc_sh, gsems, ssems):
    cid = lax.axis_index("c")
    sid = lax.axis_index("s")
    abase = sid * rows_per_tile

    def fill_rows0(val):
      @pl.loop(0, C)
      def _(r):
        for cc in range(d_feat // L):
          rows0[r, pl.ds(cc * L, L)] = jnp.full((L,), val, jnp.float32)

    def zero_acc():
      for b in range(rows_per_tile // C):
        pltpu.sync_copy(rows0, acc_sh.at[pl.ds(abase + b * C, C)])

    fill_rows0(0.0)
    zero_acc()

    plsc.subcore_barrier()

    bufs = (rows0, rows1, rows2, rows3)

    def gather_start(cix, b):
      pltpu.async_copy(x_hbm.at[src_v.at[cix]], bufs[b], gsems[b])

    def gather_wait(cix, b):
      pltpu.make_async_copy(x_hbm.at[src_v.at[cix]], bufs[b], gsems[b]).wait()

    def scatter_start(cix, b):
      pltpu.async_copy(bufs[b], acc_sh.at[dst_v.at[cix]], ssems[b], add=True)

    def scatter_wait(cix, b):
      pltpu.make_async_copy(
          bufs[b], acc_sh.at[dst_v.at[cix]], ssems[b]).wait()

    @pl.when(cid == 0)
    def _():
      for p in range(NPHASE):
        pltpu.sync_copy(dst_hbm.at[sid * NPHASE + p], dst_v)
        pltpu.sync_copy(src_hbm.at[sid * NPHASE + p], src_v)

        gather_start(0, 0)
        gather_start(1, 1)
        gather_start(2, 2)

        @pl.loop(0, NCHUNK, step=4)
        def _(j):
          for b in range(4):
            cix = j + b
            b3 = (b + 3) % 4
            nxt = cix + 3

            @pl.when(nxt < NCHUNK)
            def _():
              @pl.when(cix >= 1)
              def _():
                scatter_wait(cix - 1, b3)

              gather_start(nxt, b3)

            gather_wait(cix, b)
            scatter_start(cix, b)

        for b in range(4):
          cix = NCHUNK - 4 + b
          scatter_wait(cix, cix % 4)

      plsc.subcore_barrier()
      pltpu.sync_copy(acc_sh.at[pl.ds(abase, rows_per_tile)],
                      sums_hbm.at[pl.ds(abase, rows_per_tile)])

    @pl.when(cid == 1)
    def _():
      fill_rows0(1.0)
      for p in range(NPHASE):
        pltpu.sync_copy(dst_hbm.at[sid * NPHASE + p], dst_v)

        @pl.loop(0, NCHUNK)
        def _(j):
          pltpu.sync_copy(rows0, acc_sh.at[dst_v.at[j]], add=True)

      plsc.subcore_barrier()
      pltpu.sync_copy(acc_sh.at[pl.ds(abase, rows_per_tile)],
                      cnts_hbm.at[pl.ds(abase, rows_per_tile)])

  return k(x, src, dst)


def _finalize(agg, n_nodes, d_feat):
  rb = 1000
  grid = n_nodes // rb

  def body(s_ref, c_ref, o_ref):
    cnt = c_ref[:, 0:1]
    o_ref[...] = s_ref[...] / jnp.maximum(cnt, 1.0)

  return pl.pallas_call(
      body,
      grid=(grid,),
      in_specs=[pl.BlockSpec((rb, d_feat), lambda i: (i, 0)),
                pl.BlockSpec((rb, d_feat), lambda i: (i, 0))],
      out_specs=pl.BlockSpec((rb, d_feat), lambda i: (i, 0)),
      out_shape=jax.ShapeDtypeStruct((n_nodes, d_feat), jnp.float32),
  )(*agg)


@jax.jit
def kernel(x, edge_index):
  n_nodes, d_feat = x.shape
  n_edges = edge_index.shape[1]

  ei = edge_index.astype(jnp.int32)
  pad = E_PAD - n_edges
  src = jnp.concatenate([ei[0], jnp.zeros((pad,), jnp.int32)])
  dst = jnp.concatenate([ei[1], jnp.full((pad,), n_nodes, jnp.int32)])
  src = src.reshape(NS * NPHASE, NCHUNK, C)
  dst = dst.reshape(NS * NPHASE, NCHUNK, C)

  agg = _sc_aggregate(x, src, dst, n_nodes, d_feat)
  return _finalize(agg, n_nodes, d_feat)


# speedup vs baseline: 1.0850x; 1.0001x over previous
import functools

import jax
import jax.numpy as jnp
from jax import lax
from jax.experimental import pallas as pl
from jax.experimental.pallas import tpu as pltpu
from jax.experimental.pallas import tpu_sc as plsc

NC = 2
NS = 16
L = 16

C = 40
NPHASE = 8
NCHUNK = 64
EPT = C * NCHUNK * NPHASE
E_PAD = NS * EPT
ACC_ROWS = 10240


def _sc_aggregate(x, src, dst, n_nodes, d_feat):
  rows_per_tile = ACC_ROWS // NS

  mesh = plsc.VectorSubcoreMesh(
      core_axis_name="c", subcore_axis_name="s", num_cores=NC)

  @functools.partial(
      pl.kernel,
      mesh=mesh,
      out_type=(
          jax.ShapeDtypeStruct((ACC_ROWS, d_feat), jnp.float32),
          jax.ShapeDtypeStruct((ACC_ROWS, d_feat), jnp.float32),
      ),
      scratch_types=[
          pltpu.VMEM((NCHUNK, C), jnp.int32),
          pltpu.VMEM((NCHUNK, C), jnp.int32),
          pltpu.VMEM((C, d_feat), jnp.float32),
          pltpu.VMEM((C, d_feat), jnp.float32),
          pltpu.VMEM((C, d_feat), jnp.float32),
          pltpu.VMEM((C, d_feat), jnp.float32),
          pltpu.VMEM_SHARED((ACC_ROWS, d_feat), jnp.float32),
          [pltpu.SemaphoreType.DMA] * 4,
          [pltpu.SemaphoreType.DMA] * 4,
      ],
  )
  def k(x_hbm, src_hbm, dst_hbm, sums_hbm, cnts_hbm,
        src_v, dst_v, rows0, rows1, rows2, rows3, acc_sh, gsems, ssems):
    cid = lax.axis_index("c")
    sid = lax.axis_index("s")
    abase = sid * rows_per_tile

    def fill_rows0(val):
      @pl.loop(0, C)
      def _(r):
        for cc in range(d_feat // L):
          rows0[r, pl.ds(cc * L, L)] = jnp.full((L,), val, jnp.float32)

    def zero_acc():
      nfull = rows_per_tile // C
      for b in range(nfull):
        pltpu.sync_copy(rows0, acc_sh.at[pl.ds(abase + b * C, C)])
      rem = rows_per_tile - nfull * C
      if rem:
        pltpu.sync_copy(rows0.at[pl.ds(0, rem)],
                        acc_sh.at[pl.ds(abase + nfull * C, rem)])

    fill_rows0(0.0)
    zero_acc()

    plsc.subcore_barrier()

    bufs = (rows0, rows1, rows2, rows3)

    def gather_start(cix, b):
      pltpu.async_copy(x_hbm.at[src_v.at[cix]], bufs[b], gsems[b])

    def gather_wait(cix, b):
      pltpu.make_async_copy(x_hbm.at[src_v.at[cix]], bufs[b], gsems[b]).wait()

    def scatter_start(cix, b):
      pltpu.async_copy(bufs[b], acc_sh.at[dst_v.at[cix]], ssems[b], add=True)

    def scatter_wait(cix, b):
      pltpu.make_async_copy(
          bufs[b], acc_sh.at[dst_v.at[cix]], ssems[b]).wait()

    @pl.when(cid == 0)
    def _():
      for p in range(NPHASE):
        pltpu.sync_copy(dst_hbm.at[sid * NPHASE + p], dst_v)
        pltpu.sync_copy(src_hbm.at[sid * NPHASE + p], src_v)

        gather_start(0, 0)
        gather_start(1, 1)
        gather_start(2, 2)

        @pl.loop(0, NCHUNK, step=4)
        def _(j):
          for b in range(4):
            cix = j + b
            b3 = (b + 3) % 4
            nxt = cix + 3

            @pl.when(nxt < NCHUNK)
            def _():
              @pl.when(cix >= 1)
              def _():
                scatter_wait(cix - 1, b3)

              gather_start(nxt, b3)

            gather_wait(cix, b)
            scatter_start(cix, b)

        for b in range(4):
          cix = NCHUNK - 4 + b
          scatter_wait(cix, cix % 4)

      plsc.subcore_barrier()
      pltpu.sync_copy(acc_sh.at[pl.ds(abase, rows_per_tile)],
                      sums_hbm.at[pl.ds(abase, rows_per_tile)])

    @pl.when(cid == 1)
    def _():
      fill_rows0(1.0)
      for p in range(NPHASE):
        pltpu.sync_copy(dst_hbm.at[sid * NPHASE + p], dst_v)

        @pl.loop(0, NCHUNK)
        def _(j):
          pltpu.sync_copy(rows0, acc_sh.at[dst_v.at[j]], add=True)

      plsc.subcore_barrier()
      pltpu.sync_copy(acc_sh.at[pl.ds(abase, rows_per_tile)],
                      cnts_hbm.at[pl.ds(abase, rows_per_tile)])

  return k(x, src, dst)


def _finalize(agg, n_nodes, d_feat):
  rb = 1000
  grid = n_nodes // rb

  def body(s_ref, c_ref, o_ref):
    cnt = c_ref[:, 0:1]
    o_ref[...] = s_ref[...] / jnp.maximum(cnt, 1.0)

  return pl.pallas_call(
      body,
      grid=(grid,),
      in_specs=[pl.BlockSpec((rb, d_feat), lambda i: (i, 0)),
                pl.BlockSpec((rb, d_feat), lambda i: (i, 0))],
      out_specs=pl.BlockSpec((rb, d_feat), lambda i: (i, 0)),
      out_shape=jax.ShapeDtypeStruct((n_nodes, d_feat), jnp.float32),
  )(*agg)


@jax.jit
def kernel(x, edge_index):
  n_nodes, d_feat = x.shape
  n_edges = edge_index.shape[1]

  ei = edge_index.astype(jnp.int32)
  pad = E_PAD - n_edges
  src = jnp.concatenate([ei[0], jnp.zeros((pad,), jnp.int32)])
  dst = jnp.concatenate([ei[1], jnp.full((pad,), n_nodes, jnp.int32)])
  src = src.reshape(NS * NPHASE, NCHUNK, C)
  dst = dst.reshape(NS * NPHASE, NCHUNK, C)

  agg = _sc_aggregate(x, src, dst, n_nodes, d_feat)
  return _finalize(agg, n_nodes, d_feat)
